# baseline (speedup 1.0000x reference)
import jax
import jax.numpy as jnp
from jax import lax
from jax.experimental import pallas as pl
from jax.experimental.pallas import tpu as pltpu
from jax.experimental.pallas import tpu_sc as plsc

NE = 768
BATCH = 4
T = 1024
NW = 32
POS_PER_W = T // NW
ROWS_PER_W = BATCH * POS_PER_W
CHUNK = 16
NCHUNK = ROWS_PER_W // CHUNK
LANES = 16


def _emb_body(x_hbm, wpe_hbm, wte_hbm, out_hbm,
              tok0, tok1, tok2, wpe_lo, wpe_hi, idx_all,
              isem, psem, gsem0, gsem1, gsem2, osem0, osem1, osem2):
    c = lax.axis_index("c")
    s = lax.axis_index("s")
    wid = s * 2 + c
    tbase = wid * POS_PER_W

    icopies = [
        pltpu.async_copy(x_hbm.at[pl.ds(b * T + tbase, POS_PER_W)],
                         idx_all.at[pl.ds(b * POS_PER_W, POS_PER_W)], isem)
        for b in range(BATCH)
    ]
    wcopies = [
        pltpu.async_copy(wpe_hbm.at[pl.ds(tbase + h * CHUNK, CHUNK), :],
                         (wpe_lo, wpe_hi)[h], psem)
        for h in range(2)
    ]
    for cp in icopies:
        cp.wait()

    toks = (tok0, tok1, tok2)
    gsems = (gsem0, gsem1, gsem2)
    osems = (osem0, osem1, osem2)

    def issue_gather(ci):
        p = ci % 3
        return pltpu.async_copy(
            wte_hbm.at[idx_all.at[pl.ds(ci * CHUNK, CHUNK)]], toks[p], gsems[p])

    gathers = {ci: issue_gather(ci) for ci in range(3)}
    for cp in wcopies:
        cp.wait()

    stores = {}
    for ci in range(NCHUNK):
        p = ci % 3
        b, h = divmod(ci, 2)
        nxt = ci + 1
        if 3 <= nxt < NCHUNK:
            with jax.named_scope(f"sw{nxt - 3}"):
                stores.pop(nxt - 3).wait()
            gathers[nxt] = issue_gather(nxt)
        with jax.named_scope(f"gw{ci}"):
            gathers.pop(ci).wait()
        tok = toks[p]
        wv = (wpe_lo, wpe_hi)[h]

        def add_row(r, carry):
            def add_jb(jb, c2):
                for j in range(16):
                    sl = pl.ds(jb * 16 * LANES + j * LANES, LANES)
                    tok[r, sl] = tok[r, sl] + wv[r, sl]
                return c2
            return lax.fori_loop(0, NE // (16 * LANES), add_jb, carry)

        with jax.named_scope(f"add{ci}"):
            lax.fori_loop(0, CHUNK, add_row, 0)
        stores[ci] = pltpu.async_copy(
            tok, out_hbm.at[pl.ds(b * T + tbase + h * CHUNK, CHUNK), :],
            osems[p])
    for ci in list(stores):
        stores.pop(ci).wait()


@jax.jit
def _embedding(x_flat, wpe, wte):
    mesh = plsc.VectorSubcoreMesh(core_axis_name="c", subcore_axis_name="s")
    run = pl.kernel(
        _emb_body,
        out_type=jax.ShapeDtypeStruct((BATCH * T, NE), jnp.float32),
        mesh=mesh,
        scratch_types=[
            pltpu.VMEM((CHUNK, NE), jnp.float32),
            pltpu.VMEM((CHUNK, NE), jnp.float32),
            pltpu.VMEM((CHUNK, NE), jnp.float32),
            pltpu.VMEM((CHUNK, NE), jnp.float32),
            pltpu.VMEM((CHUNK, NE), jnp.float32),
            pltpu.VMEM((ROWS_PER_W,), jnp.int32),
            pltpu.SemaphoreType.DMA,
            pltpu.SemaphoreType.DMA,
            pltpu.SemaphoreType.DMA,
            pltpu.SemaphoreType.DMA,
            pltpu.SemaphoreType.DMA,
            pltpu.SemaphoreType.DMA,
            pltpu.SemaphoreType.DMA,
            pltpu.SemaphoreType.DMA,
        ],
    )
    return run(x_flat, wpe, wte)


def kernel(x, wte, wpe):
    b, t = x.shape
    x_flat = x.reshape(b * t).astype(jnp.int32)
    out = _embedding(x_flat, wpe, wte)
    return out.reshape(b, t, NE)


# speedup vs baseline: 1.6402x; 1.6402x over previous
import jax
import jax.numpy as jnp
from jax import lax
from jax.experimental import pallas as pl
from jax.experimental.pallas import tpu as pltpu
from jax.experimental.pallas import tpu_sc as plsc

NE = 768
BATCH = 4
T = 1024
NW = 32
POS_PER_W = T // NW
ROWS_PER_W = BATCH * POS_PER_W
CHUNK = 16
NCHUNK = ROWS_PER_W // CHUNK
LANES = 16


def _emb_body(x_hbm, wpe_hbm, wte_hbm, out_hbm,
              tok0, tok1, tok2, wpe_lo, wpe_hi, idx_all,
              isem, psem, gsem0, gsem1, gsem2, osem0, osem1, osem2):
    c = lax.axis_index("c")
    s = lax.axis_index("s")
    wid = s * 2 + c
    tbase = wid * POS_PER_W

    icopies = [
        pltpu.async_copy(x_hbm.at[b, pl.ds(tbase, POS_PER_W)],
                         idx_all.at[pl.ds(b * POS_PER_W, POS_PER_W)], isem)
        for b in range(BATCH)
    ]
    wcopies = [
        pltpu.async_copy(wpe_hbm.at[pl.ds(tbase + h * CHUNK, CHUNK), :],
                         (wpe_lo, wpe_hi)[h], psem)
        for h in range(2)
    ]
    for cp in icopies:
        cp.wait()

    toks = (tok0, tok1, tok2)
    gsems = (gsem0, gsem1, gsem2)
    osems = (osem0, osem1, osem2)

    def issue_gather(ci):
        p = ci % 3
        return pltpu.async_copy(
            wte_hbm.at[idx_all.at[pl.ds(ci * CHUNK, CHUNK)]], toks[p], gsems[p])

    gathers = {ci: issue_gather(ci) for ci in range(3)}
    for cp in wcopies:
        cp.wait()

    stores = {}
    for ci in range(NCHUNK):
        p = ci % 3
        b, h = divmod(ci, 2)
        nxt = ci + 1
        if 3 <= nxt < NCHUNK:
            with jax.named_scope(f"sw{nxt - 3}"):
                stores.pop(nxt - 3).wait()
            gathers[nxt] = issue_gather(nxt)
        with jax.named_scope(f"gw{ci}"):
            gathers.pop(ci).wait()
        tok = toks[p]
        wv = (wpe_lo, wpe_hi)[h]

        def add_row(r, carry):
            for j in range(NE // LANES):
                sl = pl.ds(j * LANES, LANES)
                tok[r, sl] = tok[r, sl] + wv[r, sl]
            return carry

        with jax.named_scope(f"add{ci}"):
            lax.fori_loop(0, CHUNK, add_row, 0)
        stores[ci] = pltpu.async_copy(
            tok, out_hbm.at[pl.ds(b * T + tbase + h * CHUNK, CHUNK), :],
            osems[p])
    for ci in list(stores):
        stores.pop(ci).wait()


@jax.jit
def _embedding(x2d, wpe, wte):
    mesh = plsc.VectorSubcoreMesh(core_axis_name="c", subcore_axis_name="s")
    run = pl.kernel(
        _emb_body,
        out_type=jax.ShapeDtypeStruct((BATCH * T, NE), jnp.float32),
        mesh=mesh,
        scratch_types=[
            pltpu.VMEM((CHUNK, NE), jnp.float32),
            pltpu.VMEM((CHUNK, NE), jnp.float32),
            pltpu.VMEM((CHUNK, NE), jnp.float32),
            pltpu.VMEM((CHUNK, NE), jnp.float32),
            pltpu.VMEM((CHUNK, NE), jnp.float32),
            pltpu.VMEM((ROWS_PER_W,), jnp.int32),
            pltpu.SemaphoreType.DMA,
            pltpu.SemaphoreType.DMA,
            pltpu.SemaphoreType.DMA,
            pltpu.SemaphoreType.DMA,
            pltpu.SemaphoreType.DMA,
            pltpu.SemaphoreType.DMA,
            pltpu.SemaphoreType.DMA,
            pltpu.SemaphoreType.DMA,
        ],
    )
    return run(x2d, wpe, wte)


def kernel(x, wte, wpe):
    b, t = x.shape
    out = _embedding(x.astype(jnp.int32), wpe, wte)
    return out.reshape(b, t, NE)


# speedup vs baseline: 1.6637x; 1.0144x over previous
import jax
import jax.numpy as jnp
from jax import lax
from jax.experimental import pallas as pl
from jax.experimental.pallas import tpu as pltpu
from jax.experimental.pallas import tpu_sc as plsc

NE = 768
BATCH = 4
T = 1024
NW = 32
POS_PER_W = T // NW
ROWS_PER_W = BATCH * POS_PER_W
CHUNK = 16
NCHUNK = ROWS_PER_W // CHUNK
LANES = 16


def _emb_body(x_hbm, wpe_hbm, wte_hbm, out_hbm,
              tok0, tok1, tok2, wpe_lo, wpe_hi, idx_all,
              isem, psem, gsem0, gsem1, gsem2, osem0, osem1, osem2):
    c = lax.axis_index("c")
    s = lax.axis_index("s")
    wid = s * 2 + c
    tbase = wid * POS_PER_W

    icopies = [
        pltpu.async_copy(x_hbm.at[b, pl.ds(tbase, POS_PER_W)],
                         idx_all.at[pl.ds(b * POS_PER_W, POS_PER_W)], isem)
        for b in range(BATCH)
    ]
    wcopies = [
        pltpu.async_copy(wpe_hbm.at[pl.ds(tbase + h * CHUNK, CHUNK), :],
                         (wpe_lo, wpe_hi)[h], psem)
        for h in range(2)
    ]
    toks = (tok0, tok1, tok2)
    gsems = (gsem0, gsem1, gsem2)
    osems = (osem0, osem1, osem2)

    idx_ready = [False] * BATCH

    def issue_gather(ci):
        p = ci % 3
        b = ci // 2
        if not idx_ready[b]:
            icopies[b].wait()
            idx_ready[b] = True
        return pltpu.async_copy(
            wte_hbm.at[idx_all.at[pl.ds(ci * CHUNK, CHUNK)]], toks[p], gsems[p])

    gathers = {ci: issue_gather(ci) for ci in range(3)}
    for cp in wcopies:
        cp.wait()

    stores = {}
    for ci in range(NCHUNK):
        p = ci % 3
        b, h = divmod(ci, 2)
        nxt = ci + 1
        if 3 <= nxt < NCHUNK:
            stores.pop(nxt - 3).wait()
            gathers[nxt] = issue_gather(nxt)
        gathers.pop(ci).wait()
        tok = toks[p]
        wv = (wpe_lo, wpe_hi)[h]

        def add_row(r, carry):
            for j in range(NE // LANES):
                sl = pl.ds(j * LANES, LANES)
                tok[r, sl] = tok[r, sl] + wv[r, sl]
            return carry

        lax.fori_loop(0, CHUNK, add_row, 0)
        stores[ci] = pltpu.async_copy(
            tok, out_hbm.at[pl.ds(b * T + tbase + h * CHUNK, CHUNK), :],
            osems[p])
    for ci in list(stores):
        stores.pop(ci).wait()


@jax.jit
def _embedding(x2d, wpe, wte):
    mesh = plsc.VectorSubcoreMesh(core_axis_name="c", subcore_axis_name="s")
    run = pl.kernel(
        _emb_body,
        out_type=jax.ShapeDtypeStruct((BATCH * T, NE), jnp.float32),
        mesh=mesh,
        scratch_types=[
            pltpu.VMEM((CHUNK, NE), jnp.float32),
            pltpu.VMEM((CHUNK, NE), jnp.float32),
            pltpu.VMEM((CHUNK, NE), jnp.float32),
            pltpu.VMEM((CHUNK, NE), jnp.float32),
            pltpu.VMEM((CHUNK, NE), jnp.float32),
            pltpu.VMEM((ROWS_PER_W,), jnp.int32),
            pltpu.SemaphoreType.DMA,
            pltpu.SemaphoreType.DMA,
            pltpu.SemaphoreType.DMA,
            pltpu.SemaphoreType.DMA,
            pltpu.SemaphoreType.DMA,
            pltpu.SemaphoreType.DMA,
            pltpu.SemaphoreType.DMA,
            pltpu.SemaphoreType.DMA,
        ],
    )
    return run(x2d, wpe, wte)


def kernel(x, wte, wpe):
    b, t = x.shape
    out = _embedding(x.astype(jnp.int32), wpe, wte)
    return out.reshape(b, t, NE)
